# baseline (speedup 1.0000x reference)
import functools

import jax
import jax.numpy as jnp
from jax import lax
from jax.experimental import pallas as pl
from jax.experimental.pallas import tpu as pltpu
from jax.experimental.pallas import tpu_sc as plsc

ENT_TOT = 1000000
REL_TOT = 1000
DIM = 64
B = 16384

NC = 2
NS = 16
L = 16
NW = NC * NS
BPW = B // NW
CH = 128
NCHUNK = BPW // CH
GP = CH // L
PDIM = 2 * DIM


def _nrsqrt(x):
    xi = plsc.bitcast(x, jnp.int32)
    yi = jnp.int32(0x5F3759DF) - (xi >> 1)
    y = plsc.bitcast(yi, jnp.float32)
    half = x * jnp.float32(-0.5)
    for _ in range(3):
        y = y * (jnp.float32(1.5) + half * y * y)
    return y


def _scores_kernel(head_hbm, rel_hbm, tail_hbm, neg_hbm,
                   hcol_hbm, rcol_hbm, tcol_hbm, ncol_hbm,
                   entp_hbm, relp_hbm,
                   pos_out, neg_out,
                   ih2, ir2, it2, in2, ihc, irc, itc, inc,
                   hv, rv, tv, nv,
                   ps1, ps2, ns1, ns2, sem):
    wid = lax.axis_index("s") * NC + lax.axis_index("c")
    base = wid * BPW

    row_iota = lax.iota(jnp.int32, L)

    for c in range(NCHUNK):
        cb = base + c * CH
        pltpu.sync_copy(head_hbm.at[pl.ds(cb, CH)], ih2)
        pltpu.sync_copy(rel_hbm.at[pl.ds(cb, CH)], ir2)
        pltpu.sync_copy(tail_hbm.at[pl.ds(cb, CH)], it2)
        pltpu.sync_copy(neg_hbm.at[pl.ds(cb, CH)], in2)
        pltpu.sync_copy(hcol_hbm.at[pl.ds(cb, CH)], ihc)
        pltpu.sync_copy(rcol_hbm.at[pl.ds(cb, CH)], irc)
        pltpu.sync_copy(tcol_hbm.at[pl.ds(cb, CH)], itc)
        pltpu.sync_copy(ncol_hbm.at[pl.ds(cb, CH)], inc)

        cp1 = pltpu.async_copy(entp_hbm.at[ih2], hv, sem)
        cp2 = pltpu.async_copy(relp_hbm.at[ir2], rv, sem)
        cp3 = pltpu.async_copy(entp_hbm.at[it2], tv, sem)
        cp4 = pltpu.async_copy(entp_hbm.at[in2], nv, sem)
        cp1.wait()
        cp2.wait()
        cp3.wait()
        cp4.wait()

        def group_body(g, _):
            rows = row_iota + g * L
            s = pl.ds(g * L, L)
            hpar = ihc[s]
            rpar = irc[s]
            tpar = itc[s]
            npar = inc[s]
            zero = jnp.zeros((L,), jnp.float32)

            def d_body(d, carry):
                hh, tt, nn, sa, sb, sc_, sd = carry
                h = plsc.load_gather(hv, [rows, hpar + d])
                r = plsc.load_gather(rv, [rows, rpar + d])
                t = plsc.load_gather(tv, [rows, tpar + d])
                n = plsc.load_gather(nv, [rows, npar + d])
                rt = r * t
                hrt = h * rt
                nrt = n * rt
                hrn = h * r * n
                hh = hh + h * h
                tt = tt + t * t
                nn = nn + n * n
                sa = sa + hrt
                sb = sb + hrt * hrt
                sc_ = sc_ + nrt
                sd = sd + hrn * hrn
                return (hh, tt, nn, sa, sb, sc_, sd)

            hh, tt, nn, sa, sb, sc_, sd = lax.fori_loop(
                0, DIM, d_body, (zero,) * 7)

            big = jnp.float32(1e12)
            inv_h = jnp.minimum(_nrsqrt(hh), big)
            inv_t = jnp.minimum(_nrsqrt(tt), big)
            inv_n = jnp.minimum(_nrsqrt(nn), big)
            norm_b = sb * _nrsqrt(sb)
            norm_d = sd * _nrsqrt(sd)
            ht = inv_h * inv_t
            off = c * CH + g * L
            ps1[pl.ds(off, L)] = -(sa * ht)
            ps2[pl.ds(off, L)] = -(norm_b * ht)
            ns1[pl.ds(off, L)] = -(sc_ * inv_n * inv_t)
            ns2[pl.ds(off, L)] = -(norm_d * inv_h * inv_n)
            return 0

        lax.fori_loop(0, GP, group_body, 0)

    pltpu.sync_copy(ps1, pos_out.at[pl.ds(base, BPW)])
    pltpu.sync_copy(ps2, pos_out.at[pl.ds(B + base, BPW)])
    pltpu.sync_copy(ns1, neg_out.at[pl.ds(base, BPW)])
    pltpu.sync_copy(ns2, neg_out.at[pl.ds(B + base, BPW)])


_sc_call = functools.partial(
    pl.kernel,
    out_type=(
        jax.ShapeDtypeStruct((2 * B,), jnp.float32),
        jax.ShapeDtypeStruct((2 * B,), jnp.float32),
    ),
    mesh=plsc.VectorSubcoreMesh(core_axis_name="c", subcore_axis_name="s"),
    compiler_params=pltpu.CompilerParams(
        needs_layout_passes=False, use_tc_tiling_on_sc=False),
    scratch_types=[
        pltpu.VMEM((CH,), jnp.int32),
        pltpu.VMEM((CH,), jnp.int32),
        pltpu.VMEM((CH,), jnp.int32),
        pltpu.VMEM((CH,), jnp.int32),
        pltpu.VMEM((CH,), jnp.int32),
        pltpu.VMEM((CH,), jnp.int32),
        pltpu.VMEM((CH,), jnp.int32),
        pltpu.VMEM((CH,), jnp.int32),
        pltpu.VMEM((CH, PDIM), jnp.float32),
        pltpu.VMEM((CH, PDIM), jnp.float32),
        pltpu.VMEM((CH, PDIM), jnp.float32),
        pltpu.VMEM((CH, PDIM), jnp.float32),
        pltpu.VMEM((BPW,), jnp.float32),
        pltpu.VMEM((BPW,), jnp.float32),
        pltpu.VMEM((BPW,), jnp.float32),
        pltpu.VMEM((BPW,), jnp.float32),
        pltpu.SemaphoreType.DMA,
    ],
)(_scores_kernel)


def _pair_body(lo_ref, hi_ref, dst_ref):
    dst_ref[...] = jnp.concatenate([lo_ref[...].T, hi_ref[...].T], axis=1)


ESPLIT = 1 << 19
RSPLIT = 1 << 9
_TW = 4096


def _make_pair_transpose(split, tw, n_cols):
    nblk = split // tw
    last = (n_cols - 1) // tw

    return pl.pallas_call(
        _pair_body,
        grid=(nblk,),
        in_specs=[
            pl.BlockSpec((DIM, tw), lambda g: (0, g)),
            pl.BlockSpec((DIM, tw), lambda g: (0, jnp.minimum(g + nblk, last))),
        ],
        out_specs=pl.BlockSpec((tw, PDIM), lambda g: (g, 0)),
        out_shape=jax.ShapeDtypeStruct((split, PDIM), jnp.float32),
    )


_ent_pair = _make_pair_transpose(ESPLIT, _TW, ENT_TOT)
_rel_pair = _make_pair_transpose(RSPLIT, RSPLIT, REL_TOT)


def kernel(batch_head, batch_rel, batch_tail, batch_negative,
           ent_embeddings, rel_embeddings):
    ent_t = ent_embeddings.T
    rel_t = rel_embeddings.T
    entp = _ent_pair(ent_t, ent_t)
    relp = _rel_pair(rel_t, rel_t)
    bh = batch_head.astype(jnp.int32)
    br = batch_rel.astype(jnp.int32)
    bt = batch_tail.astype(jnp.int32)
    bn = batch_negative.astype(jnp.int32)
    return _sc_call(
        bh & (ESPLIT - 1),
        br & (RSPLIT - 1),
        bt & (ESPLIT - 1),
        bn & (ESPLIT - 1),
        (bh >> 19) * DIM,
        (br >> 9) * DIM,
        (bt >> 19) * DIM,
        (bn >> 19) * DIM,
        entp,
        relp,
    )


# speedup vs baseline: 1.1367x; 1.1367x over previous
import functools

import jax
import jax.numpy as jnp
from jax import lax
from jax.experimental import pallas as pl
from jax.experimental.pallas import tpu as pltpu
from jax.experimental.pallas import tpu_sc as plsc

ENT_TOT = 1000000
REL_TOT = 1000
DIM = 64
B = 16384

NC = 2
NS = 16
L = 16
NW = NC * NS
BPW = B // NW
CH = 128
NCHUNK = BPW // CH
GP = CH // L
PDIM = 2 * DIM


def _nrsqrt(x):
    xi = plsc.bitcast(x, jnp.int32)
    yi = jnp.int32(0x5F3759DF) - (xi >> 1)
    y = plsc.bitcast(yi, jnp.float32)
    half = x * jnp.float32(-0.5)
    for _ in range(3):
        y = y * (jnp.float32(1.5) + half * y * y)
    return y


def _scores_kernel(head_hbm, rel_hbm, tail_hbm, neg_hbm,
                   hcol_hbm, rcol_hbm, tcol_hbm, ncol_hbm,
                   entp_hbm, relp_hbm,
                   pos_out, neg_out,
                   ih2, ir2, it2, in2, ihc, irc, itc, inc,
                   hv, rv, tv, nv,
                   ps1, ps2, ns1, ns2, sem):
    wid = lax.axis_index("s") * NC + lax.axis_index("c")
    base = wid * BPW

    row_iota = lax.iota(jnp.int32, L)

    for c in range(NCHUNK):
        cb = base + c * CH
        pltpu.sync_copy(head_hbm.at[pl.ds(cb, CH)], ih2)
        pltpu.sync_copy(rel_hbm.at[pl.ds(cb, CH)], ir2)
        pltpu.sync_copy(tail_hbm.at[pl.ds(cb, CH)], it2)
        pltpu.sync_copy(neg_hbm.at[pl.ds(cb, CH)], in2)
        pltpu.sync_copy(hcol_hbm.at[pl.ds(cb, CH)], ihc)
        pltpu.sync_copy(rcol_hbm.at[pl.ds(cb, CH)], irc)
        pltpu.sync_copy(tcol_hbm.at[pl.ds(cb, CH)], itc)
        pltpu.sync_copy(ncol_hbm.at[pl.ds(cb, CH)], inc)

        cp1 = pltpu.async_copy(entp_hbm.at[ih2], hv, sem)
        cp2 = pltpu.async_copy(relp_hbm.at[ir2], rv, sem)
        cp3 = pltpu.async_copy(entp_hbm.at[it2], tv, sem)
        cp4 = pltpu.async_copy(entp_hbm.at[in2], nv, sem)
        cp1.wait()
        cp2.wait()
        cp3.wait()
        cp4.wait()

        def group_body(g, _):
            rows = row_iota + g * L
            s = pl.ds(g * L, L)
            hpar = ihc[s]
            rpar = irc[s]
            tpar = itc[s]
            npar = inc[s]
            zero = jnp.zeros((L,), jnp.float32)

            def d_body(d, carry):
                hh, tt, nn, sa, sb, sc_, sd = carry
                h = plsc.load_gather(hv, [rows, hpar + d])
                r = plsc.load_gather(rv, [rows, rpar + d])
                t = plsc.load_gather(tv, [rows, tpar + d])
                n = plsc.load_gather(nv, [rows, npar + d])
                rt = r * t
                hrt = h * rt
                nrt = n * rt
                hrn = h * r * n
                hh = hh + h * h
                tt = tt + t * t
                nn = nn + n * n
                sa = sa + hrt
                sb = sb + hrt * hrt
                sc_ = sc_ + nrt
                sd = sd + hrn * hrn
                return (hh, tt, nn, sa, sb, sc_, sd)

            hh, tt, nn, sa, sb, sc_, sd = lax.fori_loop(
                0, DIM, d_body, (zero,) * 7, unroll=8)

            big = jnp.float32(1e12)
            inv_h = jnp.minimum(_nrsqrt(hh), big)
            inv_t = jnp.minimum(_nrsqrt(tt), big)
            inv_n = jnp.minimum(_nrsqrt(nn), big)
            norm_b = sb * _nrsqrt(sb)
            norm_d = sd * _nrsqrt(sd)
            ht = inv_h * inv_t
            off = c * CH + g * L
            ps1[pl.ds(off, L)] = -(sa * ht)
            ps2[pl.ds(off, L)] = -(norm_b * ht)
            ns1[pl.ds(off, L)] = -(sc_ * inv_n * inv_t)
            ns2[pl.ds(off, L)] = -(norm_d * inv_h * inv_n)
            return 0

        lax.fori_loop(0, GP, group_body, 0)

    pltpu.sync_copy(ps1, pos_out.at[pl.ds(base, BPW)])
    pltpu.sync_copy(ps2, pos_out.at[pl.ds(B + base, BPW)])
    pltpu.sync_copy(ns1, neg_out.at[pl.ds(base, BPW)])
    pltpu.sync_copy(ns2, neg_out.at[pl.ds(B + base, BPW)])


_sc_call = functools.partial(
    pl.kernel,
    out_type=(
        jax.ShapeDtypeStruct((2 * B,), jnp.float32),
        jax.ShapeDtypeStruct((2 * B,), jnp.float32),
    ),
    mesh=plsc.VectorSubcoreMesh(core_axis_name="c", subcore_axis_name="s"),
    compiler_params=pltpu.CompilerParams(
        needs_layout_passes=False, use_tc_tiling_on_sc=False),
    scratch_types=[
        pltpu.VMEM((CH,), jnp.int32),
        pltpu.VMEM((CH,), jnp.int32),
        pltpu.VMEM((CH,), jnp.int32),
        pltpu.VMEM((CH,), jnp.int32),
        pltpu.VMEM((CH,), jnp.int32),
        pltpu.VMEM((CH,), jnp.int32),
        pltpu.VMEM((CH,), jnp.int32),
        pltpu.VMEM((CH,), jnp.int32),
        pltpu.VMEM((CH, PDIM), jnp.float32),
        pltpu.VMEM((CH, PDIM), jnp.float32),
        pltpu.VMEM((CH, PDIM), jnp.float32),
        pltpu.VMEM((CH, PDIM), jnp.float32),
        pltpu.VMEM((BPW,), jnp.float32),
        pltpu.VMEM((BPW,), jnp.float32),
        pltpu.VMEM((BPW,), jnp.float32),
        pltpu.VMEM((BPW,), jnp.float32),
        pltpu.SemaphoreType.DMA,
    ],
)(_scores_kernel)


def _pair_body(lo_ref, hi_ref, dst_ref):
    dst_ref[...] = jnp.concatenate([lo_ref[...].T, hi_ref[...].T], axis=1)


ESPLIT = 1 << 19
RSPLIT = 1 << 9
_TW = 8192


def _make_pair_transpose(split, tw, n_cols):
    nblk = split // tw
    last = (n_cols - 1) // tw

    return pl.pallas_call(
        _pair_body,
        grid=(nblk,),
        in_specs=[
            pl.BlockSpec((DIM, tw), lambda g: (0, g)),
            pl.BlockSpec((DIM, tw), lambda g: (0, jnp.minimum(g + nblk, last))),
        ],
        out_specs=pl.BlockSpec((tw, PDIM), lambda g: (g, 0)),
        out_shape=jax.ShapeDtypeStruct((split, PDIM), jnp.float32),
    )


_ent_pair = _make_pair_transpose(ESPLIT, _TW, ENT_TOT)
_rel_pair = _make_pair_transpose(RSPLIT, RSPLIT, REL_TOT)


def kernel(batch_head, batch_rel, batch_tail, batch_negative,
           ent_embeddings, rel_embeddings):
    ent_t = ent_embeddings.T
    rel_t = rel_embeddings.T
    entp = _ent_pair(ent_t, ent_t)
    relp = _rel_pair(rel_t, rel_t)
    bh = batch_head.astype(jnp.int32)
    br = batch_rel.astype(jnp.int32)
    bt = batch_tail.astype(jnp.int32)
    bn = batch_negative.astype(jnp.int32)
    return _sc_call(
        bh & (ESPLIT - 1),
        br & (RSPLIT - 1),
        bt & (ESPLIT - 1),
        bn & (ESPLIT - 1),
        (bh >> 19) * DIM,
        (br >> 9) * DIM,
        (bt >> 19) * DIM,
        (bn >> 19) * DIM,
        entp,
        relp,
    )


# speedup vs baseline: 1.3198x; 1.1611x over previous
import functools

import jax
import jax.numpy as jnp
from jax import lax
from jax.experimental import pallas as pl
from jax.experimental.pallas import tpu as pltpu
from jax.experimental.pallas import tpu_sc as plsc

ENT_TOT = 1000000
REL_TOT = 1000
DIM = 64
B = 16384

NC = 2
NS = 16
L = 16
NW = NC * NS
BPW = B // NW
CH = 128
NCHUNK = BPW // CH
GP = CH // L
PDIM = 2 * DIM


def _nrsqrt(x):
    xi = plsc.bitcast(x, jnp.int32)
    yi = jnp.int32(0x5F3759DF) - (xi >> 1)
    y = plsc.bitcast(yi, jnp.float32)
    half = x * jnp.float32(-0.5)
    for _ in range(3):
        y = y * (jnp.float32(1.5) + half * y * y)
    return y


def _scores_kernel(head_hbm, rel_hbm, tail_hbm, neg_hbm,
                   hcol_hbm, rcol_hbm, tcol_hbm, ncol_hbm,
                   entp_hbm, relp_hbm,
                   pos_out, neg_out,
                   ih2, ir2, it2, in2, ihc, irc, itc, inc,
                   hv, rv, tv, nv,
                   ps1, ps2, ns1, ns2, sem):
    wid = lax.axis_index("s") * NC + lax.axis_index("c")
    base = wid * BPW

    row_iota = lax.iota(jnp.int32, L)

    for c in range(NCHUNK):
        cb = base + c * CH
        pltpu.sync_copy(head_hbm.at[pl.ds(cb, CH)], ih2)
        pltpu.sync_copy(rel_hbm.at[pl.ds(cb, CH)], ir2)
        pltpu.sync_copy(tail_hbm.at[pl.ds(cb, CH)], it2)
        pltpu.sync_copy(neg_hbm.at[pl.ds(cb, CH)], in2)
        pltpu.sync_copy(hcol_hbm.at[pl.ds(cb, CH)], ihc)
        pltpu.sync_copy(rcol_hbm.at[pl.ds(cb, CH)], irc)
        pltpu.sync_copy(tcol_hbm.at[pl.ds(cb, CH)], itc)
        pltpu.sync_copy(ncol_hbm.at[pl.ds(cb, CH)], inc)

        cp1 = pltpu.async_copy(entp_hbm.at[ih2], hv, sem)
        cp2 = pltpu.async_copy(relp_hbm.at[ir2], rv, sem)
        cp3 = pltpu.async_copy(entp_hbm.at[it2], tv, sem)
        cp4 = pltpu.async_copy(entp_hbm.at[in2], nv, sem)
        cp1.wait()
        cp2.wait()
        cp3.wait()
        cp4.wait()

        def group_body(g, _):
            rows = row_iota + g * L
            s = pl.ds(g * L, L)
            hpar = ihc[s]
            rpar = irc[s]
            tpar = itc[s]
            npar = inc[s]
            zero = jnp.zeros((L,), jnp.float32)

            def d_body(d, carry):
                hh, tt, nn, sa, sb, sc_, sd = carry
                dvec = (d + row_iota) & (DIM - 1)
                h = plsc.load_gather(hv, [rows, hpar + dvec])
                r = plsc.load_gather(rv, [rows, rpar + dvec])
                t = plsc.load_gather(tv, [rows, tpar + dvec])
                n = plsc.load_gather(nv, [rows, npar + dvec])
                rt = r * t
                hrt = h * rt
                nrt = n * rt
                hrn = h * r * n
                hh = hh + h * h
                tt = tt + t * t
                nn = nn + n * n
                sa = sa + hrt
                sb = sb + hrt * hrt
                sc_ = sc_ + nrt
                sd = sd + hrn * hrn
                return (hh, tt, nn, sa, sb, sc_, sd)

            hh, tt, nn, sa, sb, sc_, sd = lax.fori_loop(
                0, DIM, d_body, (zero,) * 7, unroll=8)

            big = jnp.float32(1e12)
            inv_h = jnp.minimum(_nrsqrt(hh), big)
            inv_t = jnp.minimum(_nrsqrt(tt), big)
            inv_n = jnp.minimum(_nrsqrt(nn), big)
            norm_b = sb * _nrsqrt(sb)
            norm_d = sd * _nrsqrt(sd)
            ht = inv_h * inv_t
            off = c * CH + g * L
            ps1[pl.ds(off, L)] = -(sa * ht)
            ps2[pl.ds(off, L)] = -(norm_b * ht)
            ns1[pl.ds(off, L)] = -(sc_ * inv_n * inv_t)
            ns2[pl.ds(off, L)] = -(norm_d * inv_h * inv_n)
            return 0

        lax.fori_loop(0, GP, group_body, 0)

    pltpu.sync_copy(ps1, pos_out.at[pl.ds(base, BPW)])
    pltpu.sync_copy(ps2, pos_out.at[pl.ds(B + base, BPW)])
    pltpu.sync_copy(ns1, neg_out.at[pl.ds(base, BPW)])
    pltpu.sync_copy(ns2, neg_out.at[pl.ds(B + base, BPW)])


_sc_call = functools.partial(
    pl.kernel,
    out_type=(
        jax.ShapeDtypeStruct((2 * B,), jnp.float32),
        jax.ShapeDtypeStruct((2 * B,), jnp.float32),
    ),
    mesh=plsc.VectorSubcoreMesh(core_axis_name="c", subcore_axis_name="s"),
    compiler_params=pltpu.CompilerParams(
        needs_layout_passes=False, use_tc_tiling_on_sc=False),
    scratch_types=[
        pltpu.VMEM((CH,), jnp.int32),
        pltpu.VMEM((CH,), jnp.int32),
        pltpu.VMEM((CH,), jnp.int32),
        pltpu.VMEM((CH,), jnp.int32),
        pltpu.VMEM((CH,), jnp.int32),
        pltpu.VMEM((CH,), jnp.int32),
        pltpu.VMEM((CH,), jnp.int32),
        pltpu.VMEM((CH,), jnp.int32),
        pltpu.VMEM((CH, PDIM), jnp.float32),
        pltpu.VMEM((CH, PDIM), jnp.float32),
        pltpu.VMEM((CH, PDIM), jnp.float32),
        pltpu.VMEM((CH, PDIM), jnp.float32),
        pltpu.VMEM((BPW,), jnp.float32),
        pltpu.VMEM((BPW,), jnp.float32),
        pltpu.VMEM((BPW,), jnp.float32),
        pltpu.VMEM((BPW,), jnp.float32),
        pltpu.SemaphoreType.DMA,
    ],
)(_scores_kernel)


def _pair_body(lo_ref, hi_ref, dst_ref):
    dst_ref[...] = jnp.concatenate([lo_ref[...].T, hi_ref[...].T], axis=1)


ESPLIT = 1 << 19
RSPLIT = 1 << 9
_TW = 8192


def _make_pair_transpose(split, tw, n_cols):
    nblk = split // tw
    last = (n_cols - 1) // tw

    return pl.pallas_call(
        _pair_body,
        grid=(nblk,),
        in_specs=[
            pl.BlockSpec((DIM, tw), lambda g: (0, g)),
            pl.BlockSpec((DIM, tw), lambda g: (0, jnp.minimum(g + nblk, last))),
        ],
        out_specs=pl.BlockSpec((tw, PDIM), lambda g: (g, 0)),
        out_shape=jax.ShapeDtypeStruct((split, PDIM), jnp.float32),
    )


_ent_pair = _make_pair_transpose(ESPLIT, _TW, ENT_TOT)
_rel_pair = _make_pair_transpose(RSPLIT, RSPLIT, REL_TOT)


def kernel(batch_head, batch_rel, batch_tail, batch_negative,
           ent_embeddings, rel_embeddings):
    ent_t = ent_embeddings.T
    rel_t = rel_embeddings.T
    entp = _ent_pair(ent_t, ent_t)
    relp = _rel_pair(rel_t, rel_t)
    bh = batch_head.astype(jnp.int32)
    br = batch_rel.astype(jnp.int32)
    bt = batch_tail.astype(jnp.int32)
    bn = batch_negative.astype(jnp.int32)
    return _sc_call(
        bh & (ESPLIT - 1),
        br & (RSPLIT - 1),
        bt & (ESPLIT - 1),
        bn & (ESPLIT - 1),
        (bh >> 19) * DIM,
        (br >> 9) * DIM,
        (bt >> 19) * DIM,
        (bn >> 19) * DIM,
        entp,
        relp,
    )


# speedup vs baseline: 1.4814x; 1.1224x over previous
import functools

import jax
import jax.numpy as jnp
from jax import lax
from jax.experimental import pallas as pl
from jax.experimental.pallas import tpu as pltpu
from jax.experimental.pallas import tpu_sc as plsc

ENT_TOT = 1000000
REL_TOT = 1000
DIM = 64
B = 16384

NC = 2
NS = 16
L = 16
NW = NC * NS
BPW = B // NW
CH = 128
NCHUNK = BPW // CH
GP = CH // L
PDIM = 2 * DIM

EQ = 1 << 18
RQ = 1 << 8

_HI = -65536


def _nrsqrt(x):
    xi = plsc.bitcast(x, jnp.int32)
    yi = jnp.int32(0x5F3759DF) - (xi >> 1)
    y = plsc.bitcast(yi, jnp.float32)
    half = x * jnp.float32(-0.5)
    for _ in range(3):
        y = y * (jnp.float32(1.5) + half * y * y)
    return y


def _scores_kernel(head_hbm, rel_hbm, tail_hbm, neg_hbm,
                   hcol_hbm, rcol_hbm, tcol_hbm, ncol_hbm,
                   hsel_hbm, rsel_hbm, tsel_hbm, nsel_hbm,
                   entq_hbm, relq_hbm,
                   pos_out, neg_out,
                   ih2, ir2, it2, in2, ihc, irc, itc, inc,
                   ihs, irs, its, ins,
                   hv, rv, tv, nv,
                   ps1, ps2, ns1, ns2, sem):
    wid = lax.axis_index("s") * NC + lax.axis_index("c")
    base = wid * BPW

    row_iota = lax.iota(jnp.int32, L)

    for c in range(NCHUNK):
        cb = base + c * CH
        pltpu.sync_copy(head_hbm.at[pl.ds(cb, CH)], ih2)
        pltpu.sync_copy(rel_hbm.at[pl.ds(cb, CH)], ir2)
        pltpu.sync_copy(tail_hbm.at[pl.ds(cb, CH)], it2)
        pltpu.sync_copy(neg_hbm.at[pl.ds(cb, CH)], in2)
        pltpu.sync_copy(hcol_hbm.at[pl.ds(cb, CH)], ihc)
        pltpu.sync_copy(rcol_hbm.at[pl.ds(cb, CH)], irc)
        pltpu.sync_copy(tcol_hbm.at[pl.ds(cb, CH)], itc)
        pltpu.sync_copy(ncol_hbm.at[pl.ds(cb, CH)], inc)
        pltpu.sync_copy(hsel_hbm.at[pl.ds(cb, CH)], ihs)
        pltpu.sync_copy(rsel_hbm.at[pl.ds(cb, CH)], irs)
        pltpu.sync_copy(tsel_hbm.at[pl.ds(cb, CH)], its)
        pltpu.sync_copy(nsel_hbm.at[pl.ds(cb, CH)], ins)

        cp1 = pltpu.async_copy(entq_hbm.at[ih2], hv, sem)
        cp2 = pltpu.async_copy(relq_hbm.at[ir2], rv, sem)
        cp3 = pltpu.async_copy(entq_hbm.at[it2], tv, sem)
        cp4 = pltpu.async_copy(entq_hbm.at[in2], nv, sem)
        cp1.wait()
        cp2.wait()
        cp3.wait()
        cp4.wait()

        def group_body(g, _):
            rows = row_iota + g * L
            s = pl.ds(g * L, L)
            hpar = ihc[s]
            rpar = irc[s]
            tpar = itc[s]
            npar = inc[s]
            hm = ihs[s] != 0
            rm = irs[s] != 0
            tm = its[s] != 0
            nm = ins[s] != 0
            zero = jnp.zeros((L,), jnp.float32)

            def unpack(w, m):
                return plsc.bitcast(
                    jnp.where(m, w & _HI, w << 16), jnp.float32)

            def d_body(d, carry):
                hh, tt, nn, sa, sb, sc_, sd = carry
                dvec = (d + row_iota) & (DIM - 1)
                h = unpack(plsc.load_gather(hv, [rows, hpar + dvec]), hm)
                r = unpack(plsc.load_gather(rv, [rows, rpar + dvec]), rm)
                t = unpack(plsc.load_gather(tv, [rows, tpar + dvec]), tm)
                n = unpack(plsc.load_gather(nv, [rows, npar + dvec]), nm)
                rt = r * t
                hrt = h * rt
                nrt = n * rt
                hrn = h * r * n
                hh = hh + h * h
                tt = tt + t * t
                nn = nn + n * n
                sa = sa + hrt
                sb = sb + hrt * hrt
                sc_ = sc_ + nrt
                sd = sd + hrn * hrn
                return (hh, tt, nn, sa, sb, sc_, sd)

            hh, tt, nn, sa, sb, sc_, sd = lax.fori_loop(
                0, DIM, d_body, (zero,) * 7, unroll=8)

            big = jnp.float32(1e12)
            inv_h = jnp.minimum(_nrsqrt(hh), big)
            inv_t = jnp.minimum(_nrsqrt(tt), big)
            inv_n = jnp.minimum(_nrsqrt(nn), big)
            norm_b = sb * _nrsqrt(sb)
            norm_d = sd * _nrsqrt(sd)
            ht = inv_h * inv_t
            off = c * CH + g * L
            ps1[pl.ds(off, L)] = -(sa * ht)
            ps2[pl.ds(off, L)] = -(norm_b * ht)
            ns1[pl.ds(off, L)] = -(sc_ * inv_n * inv_t)
            ns2[pl.ds(off, L)] = -(norm_d * inv_h * inv_n)
            return 0

        lax.fori_loop(0, GP, group_body, 0)

    pltpu.sync_copy(ps1, pos_out.at[pl.ds(base, BPW)])
    pltpu.sync_copy(ps2, pos_out.at[pl.ds(B + base, BPW)])
    pltpu.sync_copy(ns1, neg_out.at[pl.ds(base, BPW)])
    pltpu.sync_copy(ns2, neg_out.at[pl.ds(B + base, BPW)])


_sc_call = functools.partial(
    pl.kernel,
    out_type=(
        jax.ShapeDtypeStruct((2 * B,), jnp.float32),
        jax.ShapeDtypeStruct((2 * B,), jnp.float32),
    ),
    mesh=plsc.VectorSubcoreMesh(core_axis_name="c", subcore_axis_name="s"),
    compiler_params=pltpu.CompilerParams(
        needs_layout_passes=False, use_tc_tiling_on_sc=False),
    scratch_types=(
        [pltpu.VMEM((CH,), jnp.int32)] * 12
        + [pltpu.VMEM((CH, PDIM), jnp.int32)] * 4
        + [pltpu.VMEM((BPW,), jnp.float32)] * 4
        + [pltpu.SemaphoreType.DMA]
    ),
)(_scores_kernel)


def _pack2(a, b):
    ia = lax.shift_right_logical(lax.bitcast_convert_type(a, jnp.int32), 16)
    ib = lax.bitcast_convert_type(b, jnp.int32) & _HI
    return ia | ib


def _quad_body(r0, r1, r2, r3, dst_ref):
    w1 = _pack2(r0[...].T, r1[...].T)
    w2 = _pack2(r2[...].T, r3[...].T)
    dst_ref[...] = jnp.concatenate([w1, w2], axis=1)


def _make_quad_transpose(split, tw, n_cols):
    nblk = split // tw
    last = (n_cols - 1) // tw

    def imap(q):
        return lambda g: (0, jnp.minimum(g + q * nblk, last))

    return pl.pallas_call(
        _quad_body,
        grid=(nblk,),
        in_specs=[pl.BlockSpec((DIM, tw), imap(q)) for q in range(4)],
        out_specs=pl.BlockSpec((tw, PDIM), lambda g: (g, 0)),
        out_shape=jax.ShapeDtypeStruct((split, PDIM), jnp.int32),
    )


_TW = 8192

_ent_quad = _make_quad_transpose(EQ, _TW, ENT_TOT)
_rel_quad = _make_quad_transpose(RQ, RQ, REL_TOT)


def kernel(batch_head, batch_rel, batch_tail, batch_negative,
           ent_embeddings, rel_embeddings):
    ent_t = ent_embeddings.T
    rel_t = rel_embeddings.T
    entq = _ent_quad(ent_t, ent_t, ent_t, ent_t)
    relq = _rel_quad(rel_t, rel_t, rel_t, rel_t)
    bh = batch_head.astype(jnp.int32)
    br = batch_rel.astype(jnp.int32)
    bt = batch_tail.astype(jnp.int32)
    bn = batch_negative.astype(jnp.int32)
    return _sc_call(
        bh & (EQ - 1),
        br & (RQ - 1),
        bt & (EQ - 1),
        bn & (EQ - 1),
        (bh >> 19) * DIM,
        (br >> 9) * DIM,
        (bt >> 19) * DIM,
        (bn >> 19) * DIM,
        (bh >> 18) & 1,
        (br >> 8) & 1,
        (bt >> 18) & 1,
        (bn >> 18) & 1,
        entq,
        relq,
    )


# speedup vs baseline: 1.7806x; 1.2020x over previous
import functools

import jax
import jax.numpy as jnp
from jax import lax
from jax.experimental import pallas as pl
from jax.experimental.pallas import tpu as pltpu
from jax.experimental.pallas import tpu_sc as plsc

ENT_TOT = 1000000
REL_TOT = 1000
DIM = 64
B = 16384

NC = 2
NS = 16
L = 16
NW = NC * NS
BPW = B // NW
CH = 128
NCHUNK = BPW // CH
GP = CH // L
PDIM = 2 * DIM

EQ = 1 << 18
RQ = 1 << 8

_HI = -65536


def _nrsqrt(x):
    xi = plsc.bitcast(x, jnp.int32)
    yi = jnp.int32(0x5F3759DF) - (xi >> 1)
    y = plsc.bitcast(yi, jnp.float32)
    half = x * jnp.float32(-0.5)
    for _ in range(3):
        y = y * (jnp.float32(1.5) + half * y * y)
    return y


def _scores_kernel(head_hbm, rel_hbm, tail_hbm, neg_hbm,
                   hcol_hbm, rcol_hbm, tcol_hbm, ncol_hbm,
                   hsel_hbm, rsel_hbm, tsel_hbm, nsel_hbm,
                   entq_hbm, relq_hbm,
                   pos_out, neg_out,
                   ih2, ir2, it2, in2, ihc, irc, itc, inc,
                   ihs, irs, its, ins,
                   hv, rv, tv, nv,
                   ps1, ps2, ns1, ns2, sem):
    wid = lax.axis_index("s") * NC + lax.axis_index("c")
    base = wid * BPW

    row_iota = lax.iota(jnp.int32, L)

    for c in range(NCHUNK):
        cb = base + c * CH
        pltpu.sync_copy(head_hbm.at[pl.ds(cb, CH)], ih2)
        pltpu.sync_copy(rel_hbm.at[pl.ds(cb, CH)], ir2)
        pltpu.sync_copy(tail_hbm.at[pl.ds(cb, CH)], it2)
        pltpu.sync_copy(neg_hbm.at[pl.ds(cb, CH)], in2)
        pltpu.sync_copy(hcol_hbm.at[pl.ds(cb, CH)], ihc)
        pltpu.sync_copy(rcol_hbm.at[pl.ds(cb, CH)], irc)
        pltpu.sync_copy(tcol_hbm.at[pl.ds(cb, CH)], itc)
        pltpu.sync_copy(ncol_hbm.at[pl.ds(cb, CH)], inc)
        pltpu.sync_copy(hsel_hbm.at[pl.ds(cb, CH)], ihs)
        pltpu.sync_copy(rsel_hbm.at[pl.ds(cb, CH)], irs)
        pltpu.sync_copy(tsel_hbm.at[pl.ds(cb, CH)], its)
        pltpu.sync_copy(nsel_hbm.at[pl.ds(cb, CH)], ins)

        cp1 = pltpu.async_copy(entq_hbm.at[ih2], hv, sem)
        cp2 = pltpu.async_copy(relq_hbm.at[ir2], rv, sem)
        cp3 = pltpu.async_copy(entq_hbm.at[it2], tv, sem)
        cp4 = pltpu.async_copy(entq_hbm.at[in2], nv, sem)
        cp1.wait()
        cp2.wait()
        cp3.wait()
        cp4.wait()

        def group_body(g, _):
            rows = row_iota + g * L
            s = pl.ds(g * L, L)
            hpar = ihc[s]
            rpar = irc[s]
            tpar = itc[s]
            npar = inc[s]
            hm = ihs[s] != 0
            rm = irs[s] != 0
            tm = its[s] != 0
            nm = ins[s] != 0
            zero = jnp.zeros((L,), jnp.float32)

            def unpack(w, m):
                return plsc.bitcast(
                    jnp.where(m, w & _HI, w << 16), jnp.float32)

            def d_body(d, carry):
                hh, tt, nn, sa, sb, sc_, sd = carry
                dvec = (d + row_iota) & (DIM - 1)
                h = unpack(plsc.load_gather(hv, [rows, hpar + dvec]), hm)
                r = unpack(plsc.load_gather(rv, [rows, rpar + dvec]), rm)
                t = unpack(plsc.load_gather(tv, [rows, tpar + dvec]), tm)
                n = unpack(plsc.load_gather(nv, [rows, npar + dvec]), nm)
                rt = r * t
                hrt = h * rt
                nrt = n * rt
                hrn = h * r * n
                hh = hh + h * h
                tt = tt + t * t
                nn = nn + n * n
                sa = sa + hrt
                sb = sb + hrt * hrt
                sc_ = sc_ + nrt
                sd = sd + hrn * hrn
                return (hh, tt, nn, sa, sb, sc_, sd)

            hh, tt, nn, sa, sb, sc_, sd = lax.fori_loop(
                0, DIM, d_body, (zero,) * 7, unroll=8)

            big = jnp.float32(1e12)
            inv_h = jnp.minimum(_nrsqrt(hh), big)
            inv_t = jnp.minimum(_nrsqrt(tt), big)
            inv_n = jnp.minimum(_nrsqrt(nn), big)
            norm_b = sb * _nrsqrt(sb)
            norm_d = sd * _nrsqrt(sd)
            ht = inv_h * inv_t
            off = c * CH + g * L
            ps1[pl.ds(off, L)] = -(sa * ht)
            ps2[pl.ds(off, L)] = -(norm_b * ht)
            ns1[pl.ds(off, L)] = -(sc_ * inv_n * inv_t)
            ns2[pl.ds(off, L)] = -(norm_d * inv_h * inv_n)
            return 0

        lax.fori_loop(0, GP, group_body, 0)

    pltpu.sync_copy(ps1, pos_out.at[pl.ds(base, BPW)])
    pltpu.sync_copy(ps2, pos_out.at[pl.ds(B + base, BPW)])
    pltpu.sync_copy(ns1, neg_out.at[pl.ds(base, BPW)])
    pltpu.sync_copy(ns2, neg_out.at[pl.ds(B + base, BPW)])


_sc_call = functools.partial(
    pl.kernel,
    out_type=(
        jax.ShapeDtypeStruct((2 * B,), jnp.float32),
        jax.ShapeDtypeStruct((2 * B,), jnp.float32),
    ),
    mesh=plsc.VectorSubcoreMesh(core_axis_name="c", subcore_axis_name="s"),
    compiler_params=pltpu.CompilerParams(
        needs_layout_passes=False, use_tc_tiling_on_sc=False),
    scratch_types=(
        [pltpu.VMEM((CH,), jnp.int32)] * 12
        + [pltpu.VMEM((CH, PDIM), jnp.int32)] * 4
        + [pltpu.VMEM((BPW,), jnp.float32)] * 4
        + [pltpu.SemaphoreType.DMA]
    ),
)(_scores_kernel)


def _pack2(a, b):
    ia = lax.shift_right_logical(lax.bitcast_convert_type(a, jnp.int32), 16)
    ib = lax.bitcast_convert_type(b, jnp.int32) & _HI
    return ia | ib


def _quad_body(r0, r1, r2, r3, dst_ref):
    w1 = _pack2(r0[...], r1[...]).T
    w2 = _pack2(r2[...], r3[...]).T
    dst_ref[...] = jnp.concatenate([w1, w2], axis=1)


def _make_quad_transpose(split, tw, n_cols):
    nblk = split // tw
    last = (n_cols - 1) // tw

    def imap(q):
        return lambda g: (0, jnp.minimum(g + q * nblk, last))

    return pl.pallas_call(
        _quad_body,
        grid=(nblk,),
        in_specs=[pl.BlockSpec((DIM, tw), imap(q)) for q in range(4)],
        out_specs=pl.BlockSpec((tw, PDIM), lambda g: (g, 0)),
        out_shape=jax.ShapeDtypeStruct((split, PDIM), jnp.int32),
    )


_TW = 8192

_ent_quad = _make_quad_transpose(EQ, _TW, ENT_TOT)
_rel_quad = _make_quad_transpose(RQ, RQ, REL_TOT)


def kernel(batch_head, batch_rel, batch_tail, batch_negative,
           ent_embeddings, rel_embeddings):
    ent_t = ent_embeddings.T
    rel_t = rel_embeddings.T
    entq = _ent_quad(ent_t, ent_t, ent_t, ent_t)
    relq = _rel_quad(rel_t, rel_t, rel_t, rel_t)
    bh = batch_head.astype(jnp.int32)
    br = batch_rel.astype(jnp.int32)
    bt = batch_tail.astype(jnp.int32)
    bn = batch_negative.astype(jnp.int32)
    return _sc_call(
        bh & (EQ - 1),
        br & (RQ - 1),
        bt & (EQ - 1),
        bn & (EQ - 1),
        (bh >> 19) * DIM,
        (br >> 9) * DIM,
        (bt >> 19) * DIM,
        (bn >> 19) * DIM,
        (bh >> 18) & 1,
        (br >> 8) & 1,
        (bt >> 18) & 1,
        (bn >> 18) & 1,
        entq,
        relq,
    )


# speedup vs baseline: 1.9558x; 1.0984x over previous
import functools

import jax
import jax.numpy as jnp
from jax import lax
from jax.experimental import pallas as pl
from jax.experimental.pallas import tpu as pltpu
from jax.experimental.pallas import tpu_sc as plsc

ENT_TOT = 1000000
REL_TOT = 1000
DIM = 64
B = 16384

NC = 2
NS = 16
L = 16
NW = NC * NS
BPW = B // NW
CH = 128
NCHUNK = BPW // CH
GP = CH // L
PDIM = 2 * DIM

EQ = 1 << 18
RQ = 1 << 8

_HI = -65536


def _nrsqrt(x):
    xi = plsc.bitcast(x, jnp.int32)
    yi = jnp.int32(0x5F3759DF) - (xi >> 1)
    y = plsc.bitcast(yi, jnp.float32)
    half = x * jnp.float32(-0.5)
    for _ in range(3):
        y = y * (jnp.float32(1.5) + half * y * y)
    return y


def _scores_kernel(head_hbm, rel_hbm, tail_hbm, neg_hbm,
                   hcol_hbm, rcol_hbm, tcol_hbm, ncol_hbm,
                   hsel_hbm, rsel_hbm, tsel_hbm, nsel_hbm,
                   entq_hbm, relq_hbm,
                   pos_out, neg_out,
                   ih2, ir2, it2, in2, ihc, irc, itc, inc,
                   ihs, irs, its, ins,
                   hv, rv, tv, nv,
                   ps1, ps2, ns1, ns2, sem):
    wid = lax.axis_index("s") * NC + lax.axis_index("c")
    base = wid * BPW

    row_iota = lax.iota(jnp.int32, L)

    bsl = pl.ds(base, BPW)
    cps = [
        pltpu.async_copy(head_hbm.at[bsl], ih2, sem),
        pltpu.async_copy(rel_hbm.at[bsl], ir2, sem),
        pltpu.async_copy(tail_hbm.at[bsl], it2, sem),
        pltpu.async_copy(neg_hbm.at[bsl], in2, sem),
        pltpu.async_copy(hcol_hbm.at[bsl], ihc, sem),
        pltpu.async_copy(rcol_hbm.at[bsl], irc, sem),
        pltpu.async_copy(tcol_hbm.at[bsl], itc, sem),
        pltpu.async_copy(ncol_hbm.at[bsl], inc, sem),
        pltpu.async_copy(hsel_hbm.at[bsl], ihs, sem),
        pltpu.async_copy(rsel_hbm.at[bsl], irs, sem),
        pltpu.async_copy(tsel_hbm.at[bsl], its, sem),
        pltpu.async_copy(nsel_hbm.at[bsl], ins, sem),
    ]
    for cp in cps:
        cp.wait()

    for c in range(NCHUNK):
        csl = pl.ds(c * CH, CH)
        cp1 = pltpu.async_copy(entq_hbm.at[ih2.at[csl]], hv, sem)
        cp2 = pltpu.async_copy(relq_hbm.at[ir2.at[csl]], rv, sem)
        cp3 = pltpu.async_copy(entq_hbm.at[it2.at[csl]], tv, sem)
        cp4 = pltpu.async_copy(entq_hbm.at[in2.at[csl]], nv, sem)
        cp1.wait()
        cp2.wait()
        cp3.wait()
        cp4.wait()

        def group_body(g, _):
            rows = row_iota + g * L
            s = pl.ds(c * CH + g * L, L)
            hpar = ihc[s]
            rpar = irc[s]
            tpar = itc[s]
            npar = inc[s]
            hm = ihs[s] != 0
            rm = irs[s] != 0
            tm = its[s] != 0
            nm = ins[s] != 0
            zero = jnp.zeros((L,), jnp.float32)

            def unpack(w, m):
                return plsc.bitcast(
                    jnp.where(m, w & _HI, w << 16), jnp.float32)

            def d_body(d, carry):
                hh, tt, nn, sa, sb, sc_, sd = carry
                dvec = (d + row_iota) & (DIM - 1)
                h = unpack(plsc.load_gather(hv, [rows, hpar + dvec]), hm)
                r = unpack(plsc.load_gather(rv, [rows, rpar + dvec]), rm)
                t = unpack(plsc.load_gather(tv, [rows, tpar + dvec]), tm)
                n = unpack(plsc.load_gather(nv, [rows, npar + dvec]), nm)
                rt = r * t
                hrt = h * rt
                nrt = n * rt
                hrn = h * r * n
                hh = hh + h * h
                tt = tt + t * t
                nn = nn + n * n
                sa = sa + hrt
                sb = sb + hrt * hrt
                sc_ = sc_ + nrt
                sd = sd + hrn * hrn
                return (hh, tt, nn, sa, sb, sc_, sd)

            hh, tt, nn, sa, sb, sc_, sd = lax.fori_loop(
                0, DIM, d_body, (zero,) * 7, unroll=8)

            big = jnp.float32(1e12)
            inv_h = jnp.minimum(_nrsqrt(hh), big)
            inv_t = jnp.minimum(_nrsqrt(tt), big)
            inv_n = jnp.minimum(_nrsqrt(nn), big)
            norm_b = sb * _nrsqrt(sb)
            norm_d = sd * _nrsqrt(sd)
            ht = inv_h * inv_t
            off = c * CH + g * L
            ps1[pl.ds(off, L)] = -(sa * ht)
            ps2[pl.ds(off, L)] = -(norm_b * ht)
            ns1[pl.ds(off, L)] = -(sc_ * inv_n * inv_t)
            ns2[pl.ds(off, L)] = -(norm_d * inv_h * inv_n)
            return 0

        lax.fori_loop(0, GP, group_body, 0)

    pltpu.sync_copy(ps1, pos_out.at[pl.ds(base, BPW)])
    pltpu.sync_copy(ps2, pos_out.at[pl.ds(B + base, BPW)])
    pltpu.sync_copy(ns1, neg_out.at[pl.ds(base, BPW)])
    pltpu.sync_copy(ns2, neg_out.at[pl.ds(B + base, BPW)])


_sc_call = functools.partial(
    pl.kernel,
    out_type=(
        jax.ShapeDtypeStruct((2 * B,), jnp.float32),
        jax.ShapeDtypeStruct((2 * B,), jnp.float32),
    ),
    mesh=plsc.VectorSubcoreMesh(core_axis_name="c", subcore_axis_name="s"),
    compiler_params=pltpu.CompilerParams(
        needs_layout_passes=False, use_tc_tiling_on_sc=False),
    scratch_types=(
        [pltpu.VMEM((BPW,), jnp.int32)] * 12
        + [pltpu.VMEM((CH, PDIM), jnp.int32)] * 4
        + [pltpu.VMEM((BPW,), jnp.float32)] * 4
        + [pltpu.SemaphoreType.DMA]
    ),
)(_scores_kernel)


def _pack2(a, b):
    ia = lax.shift_right_logical(lax.bitcast_convert_type(a, jnp.int32), 16)
    ib = lax.bitcast_convert_type(b, jnp.int32) & _HI
    return ia | ib


def _quad_body(r0, r1, r2, r3, dst_ref):
    w1 = _pack2(r0[...], r1[...]).T
    w2 = _pack2(r2[...], r3[...]).T
    dst_ref[...] = jnp.concatenate([w1, w2], axis=1)


def _make_quad_transpose(split, tw, n_cols):
    nblk = split // tw
    last = (n_cols - 1) // tw

    def imap(q):
        return lambda g: (0, jnp.minimum(g + q * nblk, last))

    return pl.pallas_call(
        _quad_body,
        grid=(nblk,),
        in_specs=[pl.BlockSpec((DIM, tw), imap(q)) for q in range(4)],
        out_specs=pl.BlockSpec((tw, PDIM), lambda g: (g, 0)),
        out_shape=jax.ShapeDtypeStruct((split, PDIM), jnp.int32),
    )


_TW = 8192

_ent_quad = _make_quad_transpose(EQ, _TW, ENT_TOT)
_rel_quad = _make_quad_transpose(RQ, RQ, REL_TOT)


def kernel(batch_head, batch_rel, batch_tail, batch_negative,
           ent_embeddings, rel_embeddings):
    ent_t = ent_embeddings.T
    rel_t = rel_embeddings.T
    entq = _ent_quad(ent_t, ent_t, ent_t, ent_t)
    relq = _rel_quad(rel_t, rel_t, rel_t, rel_t)
    bh = batch_head.astype(jnp.int32)
    br = batch_rel.astype(jnp.int32)
    bt = batch_tail.astype(jnp.int32)
    bn = batch_negative.astype(jnp.int32)
    return _sc_call(
        bh & (EQ - 1),
        br & (RQ - 1),
        bt & (EQ - 1),
        bn & (EQ - 1),
        (bh >> 19) * DIM,
        (br >> 9) * DIM,
        (bt >> 19) * DIM,
        (bn >> 19) * DIM,
        (bh >> 18) & 1,
        (br >> 8) & 1,
        (bt >> 18) & 1,
        (bn >> 18) & 1,
        entq,
        relq,
    )
